# Initial kernel scaffold; baseline (speedup 1.0000x reference)
import jax
import jax.numpy as jnp
from jax.experimental import pallas as pl


def kernel(visit_segments, embedding_weight):
    raise NotImplementedError("write your pallas kernel here")


# speedup vs baseline: 5.0677x; 5.0677x over previous
import functools

import jax
import jax.numpy as jnp
from jax import lax
from jax.experimental import pallas as pl
from jax.experimental.pallas import tpu as pltpu
from jax.experimental.pallas import tpu_sc as plsc

B_ROWS = 16384
SEQ = 200
D = 32
NB = B_ROWS * SEQ

_NC, _NS = 2, 16
NW = _NC * _NS
PER_W = NB // NW

CB = 128
K = 16
CHUNK = K * CB
N_BLOCKS = NB // CB
BLK_PER_W = PER_W // CB
N_IT = BLK_PER_W // K


def _make_emb():
    mesh = plsc.VectorSubcoreMesh(core_axis_name="c", subcore_axis_name="s")

    @functools.partial(
        pl.kernel,
        mesh=mesh,
        out_type=jax.ShapeDtypeStruct((N_BLOCKS, CB, D), jnp.float32),
        scratch_types=[
            pltpu.VMEM((K, CB), jnp.int32),
            pltpu.VMEM((K, CB, D), jnp.float32),
            pltpu.SemaphoreType.DMA,
        ],
        compiler_params=pltpu.CompilerParams(use_tc_tiling_on_sc=False),
    )
    def emb(idx_hbm, table_hbm, out_hbm, idx_v, rows_v, sem):
        wid = lax.axis_index("s") * _NC + lax.axis_index("c")
        base_blk = wid * BLK_PER_W

        def body(i, carry):
            blk = base_blk + i * K
            pltpu.sync_copy(idx_hbm.at[pl.ds(blk, K)], idx_v)
            copies = [
                pltpu.async_copy(table_hbm.at[idx_v.at[k]], rows_v.at[k], sem)
                for k in range(K)
            ]
            for cp in copies:
                cp.wait()
            pltpu.sync_copy(rows_v, out_hbm.at[pl.ds(blk, K)])
            return carry

        lax.fori_loop(0, N_IT, body, 0)

    return emb


_emb = _make_emb()


def kernel(visit_segments, embedding_weight):
    idx = visit_segments.astype(jnp.int32).reshape(N_BLOCKS, CB)
    out = _emb(idx, embedding_weight)
    return out.reshape(B_ROWS, SEQ, D)


# speedup vs baseline: 5.0969x; 1.0058x over previous
import functools

import jax
import jax.numpy as jnp
from jax import lax
from jax.experimental import pallas as pl
from jax.experimental.pallas import tpu as pltpu
from jax.experimental.pallas import tpu_sc as plsc

B_ROWS = 16384
SEQ = 200
D = 32
NB = B_ROWS * SEQ

_NC, _NS = 2, 16
NW = _NC * _NS
PER_W = NB // NW

CB = 128
K = 8
CHUNK = K * CB
N_BLOCKS = NB // CB
BLK_PER_W = PER_W // CB
N_CHUNK = BLK_PER_W // K
NBUF = 2


def _make_emb():
    mesh = plsc.VectorSubcoreMesh(core_axis_name="c", subcore_axis_name="s")

    @functools.partial(
        pl.kernel,
        mesh=mesh,
        out_type=jax.ShapeDtypeStruct((N_BLOCKS, CB, D), jnp.float32),
        scratch_types=[
            pltpu.VMEM((NBUF, K, CB), jnp.int32),
            pltpu.VMEM((NBUF, K, CB, D), jnp.float32),
            pltpu.SemaphoreType.DMA,
            [pltpu.SemaphoreType.DMA] * NBUF,
        ],
        compiler_params=pltpu.CompilerParams(use_tc_tiling_on_sc=False),
    )
    def emb(idx_hbm, table_hbm, out_hbm, idx_v, rows_v, gsem, osems):
        wid = lax.axis_index("s") * _NC + lax.axis_index("c")
        base_blk = wid * BLK_PER_W

        def gather_chunk(chunk, b):
            blk = base_blk + chunk * K
            pltpu.sync_copy(idx_hbm.at[pl.ds(blk, K)], idx_v.at[b])
            copies = [
                pltpu.async_copy(
                    table_hbm.at[idx_v.at[b].at[k]], rows_v.at[b].at[k], gsem
                )
                for k in range(K)
            ]
            for cp in copies:
                cp.wait()

        def put_chunk(chunk, b):
            blk = base_blk + chunk * K
            pltpu.async_copy(rows_v.at[b], out_hbm.at[pl.ds(blk, K)], osems[b])

        def drain_out(b):
            pltpu.make_async_copy(
                rows_v.at[b], out_hbm.at[pl.ds(base_blk, K)], osems[b]
            ).wait()

        for b in range(NBUF):
            gather_chunk(b, b)
            put_chunk(b, b)

        def body(j, carry):
            for b in range(NBUF):
                chunk = NBUF + j * NBUF + b
                drain_out(b)
                gather_chunk(chunk, b)
                put_chunk(chunk, b)
            return carry

        lax.fori_loop(0, (N_CHUNK - NBUF) // NBUF, body, 0)

        for b in range(NBUF):
            drain_out(b)

    return emb


_emb = _make_emb()


def kernel(visit_segments, embedding_weight):
    idx = visit_segments.astype(jnp.int32).reshape(N_BLOCKS, CB)
    out = _emb(idx, embedding_weight)
    return out.reshape(B_ROWS, SEQ, D)


# speedup vs baseline: 6.8587x; 1.3457x over previous
import functools

import jax
import jax.numpy as jnp
from jax import lax
from jax.experimental import pallas as pl
from jax.experimental.pallas import tpu as pltpu
from jax.experimental.pallas import tpu_sc as plsc

B_ROWS = 16384
SEQ = 200
D = 32
NB = B_ROWS * SEQ

_NC, _NS = 2, 16
NW = _NC * _NS
PER_W = NB // NW

CB = 128
K = 8
CHUNK = K * CB
N_BLOCKS = NB // CB
BLK_PER_W = PER_W // CB
N_CHUNK = BLK_PER_W // K
NBUF = 2


def _make_emb():
    mesh = plsc.VectorSubcoreMesh(core_axis_name="c", subcore_axis_name="s")

    @functools.partial(
        pl.kernel,
        mesh=mesh,
        out_type=jax.ShapeDtypeStruct((N_BLOCKS, CB, D), jnp.float32),
        scratch_types=[
            pltpu.VMEM((NBUF, K, CB), jnp.int32),
            pltpu.VMEM((NBUF, K, CB, D), jnp.float32),
            pltpu.VMEM_SHARED((1000, D), jnp.float32),
            pltpu.SemaphoreType.DMA,
            [pltpu.SemaphoreType.DMA] * NBUF,
        ],
        compiler_params=pltpu.CompilerParams(use_tc_tiling_on_sc=False),
    )
    def emb(idx_hbm, table_hbm, out_hbm, idx_v, rows_v, table_v, gsem, osems):
        wid = lax.axis_index("s") * _NC + lax.axis_index("c")
        base_blk = wid * BLK_PER_W

        @pl.when(lax.axis_index("s") == 0)
        def _stage():
            pltpu.sync_copy(table_hbm, table_v)
        plsc.subcore_barrier()

        def gather_chunk(chunk, b):
            blk = base_blk + chunk * K
            pltpu.sync_copy(idx_hbm.at[pl.ds(blk, K)], idx_v.at[b])
            copies = [
                pltpu.async_copy(
                    table_v.at[idx_v.at[b].at[k]], rows_v.at[b].at[k], gsem
                )
                for k in range(K)
            ]
            for cp in copies:
                cp.wait()

        def put_chunk(chunk, b):
            blk = base_blk + chunk * K
            pltpu.async_copy(rows_v.at[b], out_hbm.at[pl.ds(blk, K)], osems[b])

        def drain_out(b):
            pltpu.make_async_copy(
                rows_v.at[b], out_hbm.at[pl.ds(base_blk, K)], osems[b]
            ).wait()

        for b in range(NBUF):
            gather_chunk(b, b)
            put_chunk(b, b)

        def body(j, carry):
            for b in range(NBUF):
                chunk = NBUF + j * NBUF + b
                drain_out(b)
                gather_chunk(chunk, b)
                put_chunk(chunk, b)
            return carry

        lax.fori_loop(0, (N_CHUNK - NBUF) // NBUF, body, 0)

        for b in range(NBUF):
            drain_out(b)

    return emb


_emb = _make_emb()


def kernel(visit_segments, embedding_weight):
    idx = visit_segments.astype(jnp.int32).reshape(N_BLOCKS, CB)
    out = _emb(idx, embedding_weight)
    return out.reshape(B_ROWS, SEQ, D)
